# baseline (speedup 1.0000x reference)
import functools

import jax
import jax.numpy as jnp
from jax import lax
from jax.experimental import pallas as pl
from jax.experimental.pallas import tpu as pltpu
from jax.experimental.pallas import tpu_sc as plsc

D_MODEL = 1024
SEQ = 4096

_info = plsc.get_sparse_core_info()
_NC = _info.num_cores
_ROWS_PER_C = SEQ // _NC
_CHUNK = 256
_NBUF = 4
_NCHUNK = _ROWS_PER_C // _CHUNK

_mesh = plsc.ScalarSubcoreMesh(axis_name="c")


@functools.partial(
    pl.kernel,
    mesh=_mesh,
    out_type=jax.ShapeDtypeStruct((SEQ, D_MODEL), jnp.float32),
    scratch_types=(
        [pltpu.VMEM_SHARED((_CHUNK, D_MODEL), jnp.float32) for _ in range(_NBUF)]
        + [pltpu.SemaphoreType.DMA for _ in range(2 * _NBUF)]
    ),
)
def _pe_slice_copy(pe_hbm, out_hbm, *scratch):
    bufs = scratch[:_NBUF]
    sins = scratch[_NBUF : 2 * _NBUF]
    souts = scratch[2 * _NBUF :]
    base = lax.axis_index("c") * _ROWS_PER_C

    in_h = [None] * _NBUF
    out_h = [None] * _NBUF
    for j in range(_NBUF):
        in_h[j] = pltpu.async_copy(
            pe_hbm.at[pl.ds(base + j * _CHUNK, _CHUNK)], bufs[j], sins[j]
        )
    for i in range(_NCHUNK):
        j = i % _NBUF
        in_h[j].wait()
        out_h[j] = pltpu.async_copy(
            bufs[j], out_hbm.at[pl.ds(base + i * _CHUNK, _CHUNK)], souts[j]
        )
        nxt = i + _NBUF
        if nxt < _NCHUNK:
            out_h[j].wait()
            in_h[j] = pltpu.async_copy(
                pe_hbm.at[pl.ds(base + nxt * _CHUNK, _CHUNK)], bufs[j], sins[j]
            )
    for j in range(_NBUF):
        out_h[j].wait()


def kernel(x, pe):
    del x
    return _pe_slice_copy(pe[0])[None]
